# baseline (device time: 76909 ns/iter reference)
import jax
import jax.numpy as jnp
from jax import lax
from jax.experimental import pallas as pl
from jax.experimental.pallas import tpu as pltpu

N_DEV = 8
T = 2048
D = 1024
DR = 16
CHUNK = 32


def _a2a_body(xs_ref, dest_ref, out_ref, dall_ref,
              local_sems, send_dest_sems, recv_dest_sems,
              send_data_sems, recv_data_sem, cp_sem):
    my = lax.axis_index("i")

    cp = pltpu.make_async_copy(dest_ref, dall_ref.at[my], local_sems.at[0])
    cp.start()
    dest_rdmas = []
    for d in range(1, N_DEV):
        tgt = lax.rem(my + d, N_DEV)
        r = pltpu.make_async_remote_copy(
            src_ref=dest_ref,
            dst_ref=dall_ref.at[my],
            send_sem=send_dest_sems.at[d - 1],
            recv_sem=recv_dest_sems.at[my],
            device_id=(tgt,),
            device_id_type=pl.DeviceIdType.MESH,
        )
        r.start()
        dest_rdmas.append(r)
    cp.wait()
    for r in dest_rdmas:
        r.wait_send()
    for d in range(1, N_DEV):
        src = lax.rem(my + d, N_DEV)
        pltpu.make_async_remote_copy(
            src_ref=dall_ref.at[src], dst_ref=dall_ref.at[src],
            send_sem=send_dest_sems.at[d - 1],
            recv_sem=recv_dest_sems.at[src],
            device_id=(my,), device_id_type=pl.DeviceIdType.MESH,
        ).wait_recv()

    dvals = dest_ref[...]
    dall = dall_ref[...]
    srcidx = lax.broadcasted_iota(jnp.int32, (N_DEV, DR, 128), 0)

    def pair(t):
        cnt = jnp.sum((dvals == t).astype(jnp.int32))
        soff = jnp.sum((dvals < t).astype(jnp.int32))
        roff = jnp.sum(((dall == t) & (srcidx < my)).astype(jnp.int32))
        return cnt, soff, roff

    for d in range(1, N_DEV):
        tgt = lax.rem(my + d, N_DEV)
        cnt, soff, roff = pair(tgt)

        def send_chunk(k, c, tgt=tgt, d=d, cnt=cnt, soff=soff, roff=roff):
            off = jnp.maximum(0, jnp.minimum(k * CHUNK, cnt - CHUNK))
            pltpu.make_async_remote_copy(
                src_ref=xs_ref.at[pl.ds(soff + off, CHUNK)],
                dst_ref=out_ref.at[pl.ds(roff + off, CHUNK)],
                send_sem=send_data_sems.at[d - 1],
                recv_sem=recv_data_sem,
                device_id=(tgt,),
                device_id_type=pl.DeviceIdType.MESH,
            ).start()
            return c

        lax.fori_loop(0, (cnt + CHUNK - 1) // CHUNK, send_chunk, 0)

    cnt_m, soff_m, roff_m = pair(my)

    def own_chunk(k, c):
        off = jnp.maximum(0, jnp.minimum(k * CHUNK, cnt_m - CHUNK))
        c2 = pltpu.make_async_copy(
            xs_ref.at[pl.ds(soff_m + off, CHUNK)],
            out_ref.at[pl.ds(roff_m + off, CHUNK)],
            cp_sem)
        c2.start()
        c2.wait()
        return c

    lax.fori_loop(0, (cnt_m + CHUNK - 1) // CHUNK, own_chunk, 0)

    for d in range(1, N_DEV):
        tgt = lax.rem(my + d, N_DEV)
        cnt, _, _ = pair(tgt)

        def wait_send_chunk(k, c, tgt=tgt, d=d):
            pltpu.make_async_remote_copy(
                src_ref=xs_ref.at[pl.ds(0, CHUNK)],
                dst_ref=out_ref.at[pl.ds(0, CHUNK)],
                send_sem=send_data_sems.at[d - 1],
                recv_sem=recv_data_sem,
                device_id=(tgt,),
                device_id_type=pl.DeviceIdType.MESH,
            ).wait_send()
            return c

        lax.fori_loop(0, (cnt + CHUNK - 1) // CHUNK, wait_send_chunk, 0)

    total_in = jnp.int32(0)
    for s in range(N_DEV):
        cnt_s = jnp.sum((dall[s] == my).astype(jnp.int32))
        nch_s = (cnt_s + CHUNK - 1) // CHUNK
        total_in = total_in + jnp.where(my == s, 0, nch_s)

    def wait_recv_chunk(k, c):
        pltpu.make_async_remote_copy(
            src_ref=xs_ref.at[pl.ds(0, CHUNK)],
            dst_ref=out_ref.at[pl.ds(0, CHUNK)],
            send_sem=send_data_sems.at[0],
            recv_sem=recv_data_sem,
            device_id=(my,),
            device_id_type=pl.DeviceIdType.MESH,
        ).wait_recv()
        return c

    lax.fori_loop(0, total_in, wait_recv_chunk, 0)


def kernel(x, dest):
    dest = dest.astype(jnp.int32)
    oh = (dest[:, None] == jnp.arange(N_DEV, dtype=jnp.int32)[None, :])
    oh = oh.astype(jnp.int32)
    rank = jnp.cumsum(oh, axis=0) - oh
    cnt = jnp.sum(oh, axis=0)
    soff = jnp.cumsum(cnt) - cnt
    pos = jnp.take_along_axis(soff[None, :] + rank, dest[:, None], axis=1)[:, 0]
    order = jnp.zeros((T,), jnp.int32).at[pos].set(
        jnp.arange(T, dtype=jnp.int32), unique_indices=True)
    xs = x[order].astype(jnp.bfloat16).reshape(T, 8, 128)
    d2 = dest.reshape(DR, 128)

    out = pl.pallas_call(
        _a2a_body,
        out_shape=jax.ShapeDtypeStruct((T, 8, 128), jnp.bfloat16),
        in_specs=[pl.BlockSpec(memory_space=pltpu.VMEM),
                  pl.BlockSpec(memory_space=pltpu.VMEM)],
        out_specs=pl.BlockSpec(memory_space=pltpu.VMEM),
        scratch_shapes=[
            pltpu.VMEM((N_DEV, DR, 128), jnp.int32),
            pltpu.SemaphoreType.DMA((1,)),
            pltpu.SemaphoreType.DMA((N_DEV - 1,)),
            pltpu.SemaphoreType.DMA((N_DEV,)),
            pltpu.SemaphoreType.DMA((N_DEV - 1,)),
            pltpu.SemaphoreType.DMA,
            pltpu.SemaphoreType.DMA,
        ],
    )(xs, d2)
    return out.reshape(T, D)


# device time: 62986 ns/iter; 1.2210x vs baseline; 1.2210x over previous
import jax
import jax.numpy as jnp
from jax import lax
from jax.experimental import pallas as pl
from jax.experimental.pallas import tpu as pltpu

N_DEV = 8
T = 2048
D = 1024
DR = 16
CHUNK = 32


def _a2a_body(xs_ref, dest_ref, out_ref, dall_ref,
              local_sems, send_dest_sems, recv_dest_sems,
              send_data_sems, recv_data_sem, cp_sem):
    my = lax.axis_index("i")

    cp = pltpu.make_async_copy(dest_ref, dall_ref.at[my], local_sems.at[0])
    cp.start()
    dest_rdmas = []
    for d in range(1, N_DEV):
        tgt = lax.rem(my + d, N_DEV)
        r = pltpu.make_async_remote_copy(
            src_ref=dest_ref,
            dst_ref=dall_ref.at[my],
            send_sem=send_dest_sems.at[d - 1],
            recv_sem=recv_dest_sems.at[my],
            device_id=(tgt,),
            device_id_type=pl.DeviceIdType.MESH,
        )
        r.start()
        dest_rdmas.append(r)
    cp.wait()
    for r in dest_rdmas:
        r.wait_send()
    for d in range(1, N_DEV):
        src = lax.rem(my + d, N_DEV)
        pltpu.make_async_remote_copy(
            src_ref=dall_ref.at[src], dst_ref=dall_ref.at[src],
            send_sem=send_dest_sems.at[d - 1],
            recv_sem=recv_dest_sems.at[src],
            device_id=(my,), device_id_type=pl.DeviceIdType.MESH,
        ).wait_recv()

    dvals = dest_ref[...]
    dall = dall_ref[...]
    srcidx = lax.broadcasted_iota(jnp.int32, (N_DEV, DR, 128), 0)

    def pair(t):
        cnt = jnp.sum((dvals == t).astype(jnp.int32))
        soff = jnp.sum((dvals < t).astype(jnp.int32))
        roff = jnp.sum(((dall == t) & (srcidx < my)).astype(jnp.int32))
        return cnt, soff, roff

    for d in range(1, N_DEV):
        tgt = lax.rem(my + d, N_DEV)
        cnt, soff, roff = pair(tgt)

        def send_chunk(k, c, tgt=tgt, d=d, cnt=cnt, soff=soff, roff=roff):
            off = jnp.maximum(0, jnp.minimum(k * CHUNK, cnt - CHUNK))
            pltpu.make_async_remote_copy(
                src_ref=xs_ref.at[pl.ds(soff + off, CHUNK)],
                dst_ref=out_ref.at[pl.ds(roff + off, CHUNK)],
                send_sem=send_data_sems.at[d - 1],
                recv_sem=recv_data_sem,
                device_id=(tgt,),
                device_id_type=pl.DeviceIdType.MESH,
            ).start()
            return c

        lax.fori_loop(0, (cnt + CHUNK - 1) // CHUNK, send_chunk, 0)

    cnt_m, soff_m, roff_m = pair(my)

    def own_chunk(k, c):
        off = jnp.maximum(0, jnp.minimum(k * CHUNK, cnt_m - CHUNK))
        c2 = pltpu.make_async_copy(
            xs_ref.at[pl.ds(soff_m + off, CHUNK)],
            out_ref.at[pl.ds(roff_m + off, CHUNK)],
            cp_sem)
        c2.start()
        c2.wait()
        return c

    lax.fori_loop(0, (cnt_m + CHUNK - 1) // CHUNK, own_chunk, 0)

    for d in range(1, N_DEV):
        tgt = lax.rem(my + d, N_DEV)
        cnt, _, _ = pair(tgt)

        def wait_send_chunk(k, c, tgt=tgt, d=d):
            pltpu.make_async_remote_copy(
                src_ref=xs_ref.at[pl.ds(0, CHUNK)],
                dst_ref=out_ref.at[pl.ds(0, CHUNK)],
                send_sem=send_data_sems.at[d - 1],
                recv_sem=recv_data_sem,
                device_id=(tgt,),
                device_id_type=pl.DeviceIdType.MESH,
            ).wait_send()
            return c

        lax.fori_loop(0, (cnt + CHUNK - 1) // CHUNK, wait_send_chunk, 0)

    total_in = jnp.int32(0)
    for s in range(N_DEV):
        cnt_s = jnp.sum((dall[s] == my).astype(jnp.int32))
        nch_s = (cnt_s + CHUNK - 1) // CHUNK
        total_in = total_in + jnp.where(my == s, 0, nch_s)

    def wait_recv_chunk(k, c):
        pltpu.make_async_remote_copy(
            src_ref=xs_ref.at[pl.ds(0, CHUNK)],
            dst_ref=out_ref.at[pl.ds(0, CHUNK)],
            send_sem=send_data_sems.at[0],
            recv_sem=recv_data_sem,
            device_id=(my,),
            device_id_type=pl.DeviceIdType.MESH,
        ).wait_recv()
        return c

    lax.fori_loop(0, total_in, wait_recv_chunk, 0)


def kernel(x, dest):
    dest = dest.astype(jnp.int32)
    order = jnp.argsort(dest, stable=True)
    xs = x[order].astype(jnp.bfloat16).reshape(T, 8, 128)
    d2 = dest.reshape(DR, 128)

    out = pl.pallas_call(
        _a2a_body,
        out_shape=jax.ShapeDtypeStruct((T, 8, 128), jnp.bfloat16),
        in_specs=[pl.BlockSpec(memory_space=pltpu.VMEM),
                  pl.BlockSpec(memory_space=pltpu.VMEM)],
        out_specs=pl.BlockSpec(memory_space=pltpu.VMEM),
        scratch_shapes=[
            pltpu.VMEM((N_DEV, DR, 128), jnp.int32),
            pltpu.SemaphoreType.DMA((1,)),
            pltpu.SemaphoreType.DMA((N_DEV - 1,)),
            pltpu.SemaphoreType.DMA((N_DEV,)),
            pltpu.SemaphoreType.DMA((N_DEV - 1,)),
            pltpu.SemaphoreType.DMA,
            pltpu.SemaphoreType.DMA,
        ],
    )(xs, d2)
    return out.reshape(T, D)
